# baseline (speedup 1.0000x reference)
import functools

import jax
import jax.numpy as jnp
from jax import lax
from jax.experimental import pallas as pl
from jax.experimental.pallas import tpu as pltpu
from jax.experimental.pallas import tpu_sc as plsc

B = 4
S = 8192
D = 128
NC = 2
NS = 16
NW = NC * NS
ROWS_PER_W = (B * S) // NW
CHUNK = 128
NCHUNK = ROWS_PER_W // CHUNK
LANES = 16
NGROUP = ROWS_PER_W // LANES
CHUNKS_PER_BATCH = S // ROWS_PER_W


def _body(len_hbm, table_hbm, out_hbm, len_v, idx_v, buf0, buf1, gsem, wsem):
    wid = lax.axis_index("s") * NC + lax.axis_index("c")
    b = wid // CHUNKS_PER_BATCH
    c = wid % CHUNKS_PER_BATCH
    out_base = wid * ROWS_PER_W

    pltpu.sync_copy(len_hbm.at[wid], len_v)
    lvec = len_v[...]

    for g in range(NGROUP):
        pos = lax.iota(jnp.int32, LANES) + (c * ROWS_PER_W + g * LANES)
        idx_v[pl.ds(g * LANES, LANES)] = jnp.where(pos < lvec, pos + 1, 0)

    bufs = [buf0, buf1]

    def gather(k, buf):
        return pltpu.async_copy(
            table_hbm.at[idx_v.at[pl.ds(k * CHUNK, CHUNK)]], buf, gsem)

    def write(k, buf):
        return pltpu.async_copy(
            buf, out_hbm.at[pl.ds(out_base + k * CHUNK, CHUNK)], wsem)

    g_h = [None] * NCHUNK
    w_h = [None] * NCHUNK
    g_h[0] = gather(0, bufs[0])
    for k in range(NCHUNK):
        g_h[k].wait()
        if k + 1 < NCHUNK:
            if k >= 1:
                w_h[k - 1].wait()
            g_h[k + 1] = gather(k + 1, bufs[(k + 1) % 2])
        w_h[k] = write(k, bufs[k % 2])
    w_h[NCHUNK - 2].wait()
    w_h[NCHUNK - 1].wait()


@jax.jit
def _run(lengths_bcast, table):
    k = functools.partial(
        pl.kernel,
        mesh=plsc.VectorSubcoreMesh(core_axis_name="c", subcore_axis_name="s"),
        out_type=jax.ShapeDtypeStruct((B * S, D), jnp.float32),
        scratch_types=[
            pltpu.VMEM((LANES,), jnp.int32),
            pltpu.VMEM((ROWS_PER_W,), jnp.int32),
            pltpu.VMEM((CHUNK, D), jnp.float32),
            pltpu.VMEM((CHUNK, D), jnp.float32),
            pltpu.SemaphoreType.DMA,
            pltpu.SemaphoreType.DMA,
        ],
    )(_body)
    return k(lengths_bcast, table)


def kernel(input, attention_mask, table):
    lengths_bcast = jnp.repeat(
        attention_mask[:, 0], (NW * LANES) // B).reshape(NW, LANES)
    out = _run(lengths_bcast, table)
    return out.reshape(B, S, D)


# speedup vs baseline: 33.8192x; 33.8192x over previous
import functools

import jax
import jax.numpy as jnp
from jax import lax
from jax.experimental import pallas as pl
from jax.experimental.pallas import tpu as pltpu
from jax.experimental.pallas import tpu_sc as plsc

B = 4
S = 8192
D = 128
NC = 2
NS = 16
NW = NC * NS
ROWS_PER_W = (B * S) // NW
CHUNK = 256
NCH = ROWS_PER_W // CHUNK
LANES = 16
NSUB = D // LANES
CHUNKS_PER_BATCH = S // ROWS_PER_W
TAIL_LO = S - CHUNK
BCAST = 128


def _body(len_hbm, table_hbm, out_hbm, len_v, buf0, buf1, bcast, gsem, wsem):
    wid = lax.axis_index("s") * NC + lax.axis_index("c")
    c = wid % CHUNKS_PER_BATCH
    out_base = wid * ROWS_PER_W
    p0 = c * ROWS_PER_W

    pltpu.sync_copy(len_hbm.at[pl.ds(wid * LANES, LANES)], len_v)
    lvec = len_v[...]
    L = lvec[0]

    bufs = [buf0, buf1]
    los = [p0 + k * CHUNK for k in range(NCH)]

    def read_copy(k):
        lo = los[k]
        buf = bufs[k % 2]
        full = pltpu.make_async_copy(
            table_hbm.at[pl.ds((lo + 1) * D, CHUNK * D)], buf, gsem)
        if k == NCH - 1:
            short = pltpu.make_async_copy(
                table_hbm.at[pl.ds((lo + 1) * D, (CHUNK - 1) * D)],
                buf.at[pl.ds(0, (CHUNK - 1) * D)], gsem)
            return (lo == TAIL_LO, short, full)
        return (None, None, full)

    def read_op(k, op):
        lo = los[k]
        is_tail, short, full = read_copy(k)

        @pl.when(lo < L)
        def _():
            if is_tail is None:
                getattr(full, op)()
            else:
                @pl.when(is_tail)
                def _():
                    getattr(short, op)()

                @pl.when(jnp.logical_not(is_tail))
                def _():
                    getattr(full, op)()

    pltpu.sync_copy(table_hbm.at[pl.ds(0, D)], bcast.at[pl.ds(0, D)])
    row0 = [bcast[pl.ds(u * LANES, LANES)] for u in range(NSUB)]

    read_op(0, "start")
    read_op(1, "start")

    def bcast_body(r, carry):
        for u in range(NSUB):
            bcast[pl.ds(r * D + u * LANES, LANES)] = row0[u]
        return carry
    lax.fori_loop(1, BCAST, bcast_body, 0)

    def fixup(k):
        lo = los[k]
        buf = bufs[k % 2]

        @pl.when(jnp.logical_and(lo < L, L < lo + CHUNK))
        def _():
            def body(r, carry):
                for u in range(NSUB):
                    buf[pl.ds(r * D + u * LANES, LANES)] = row0[u]
                return carry
            lax.fori_loop(L - lo, CHUNK, body, 0)

    def issue_write(k):
        lo = los[k]
        out_slc = out_hbm.at[pl.ds((out_base + k * CHUNK) * D, CHUNK * D)]

        @pl.when(lo < L)
        def _():
            pltpu.make_async_copy(bufs[k % 2], out_slc, wsem).start()

        @pl.when(lo >= L)
        def _():
            base = (out_base + k * CHUNK) * D
            for h in range(CHUNK // BCAST):
                pltpu.make_async_copy(
                    bcast,
                    out_hbm.at[pl.ds(base + h * BCAST * D, BCAST * D)],
                    wsem).start()

    def wait_write(k):
        out_slc = out_hbm.at[pl.ds((out_base + k * CHUNK) * D, CHUNK * D)]
        pltpu.make_async_copy(bufs[k % 2], out_slc, wsem).wait()

    for k in range(NCH):
        read_op(k, "wait")
        fixup(k)
        issue_write(k)
        if k + 2 < NCH:
            wait_write(k)
            read_op(k + 2, "start")
    wait_write(NCH - 2)
    wait_write(NCH - 1)


@jax.jit
def _run(lengths_bcast, table_flat):
    k = functools.partial(
        pl.kernel,
        mesh=plsc.VectorSubcoreMesh(core_axis_name="c", subcore_axis_name="s"),
        out_type=jax.ShapeDtypeStruct((B * S * D,), jnp.float32),
        scratch_types=[
            pltpu.VMEM((LANES,), jnp.int32),
            pltpu.VMEM((CHUNK * D,), jnp.float32),
            pltpu.VMEM((CHUNK * D,), jnp.float32),
            pltpu.VMEM((BCAST * D,), jnp.float32),
            pltpu.SemaphoreType.DMA,
            pltpu.SemaphoreType.DMA,
        ],
    )(_body)
    return k(lengths_bcast, table_flat)


def kernel(input, attention_mask, table):
    lengths_bcast = jnp.repeat(attention_mask[:, 0], (NW * LANES) // B)
    out = _run(lengths_bcast, table.reshape(-1))
    return out.reshape(B, S, D)
